# Initial kernel scaffold; baseline (speedup 1.0000x reference)
import jax
import jax.numpy as jnp
from jax.experimental import pallas as pl


def kernel(x, edge_index, W1, b1, W2, b2, W3, b3, W4, b4, W5, b5):
    raise NotImplementedError("write your pallas kernel here")


# speedup vs baseline: 4.5136x; 4.5136x over previous
import functools

import jax
import jax.numpy as jnp
from jax import lax
from jax.experimental import pallas as pl
from jax.experimental.pallas import tpu as pltpu
from jax.experimental.pallas import tpu_sc as plsc

N = 10000
E = 320000
NSC = 2
NTILE = 16
EB = 128
RB = 80
N_EDGE_CHUNKS = E // EB
N_ROW_CHUNKS = N // RB


def _sc_aggregate(zs, src2, dst, f2):
    mesh = plsc.VectorSubcoreMesh(core_axis_name="c", subcore_axis_name="s")

    @functools.partial(
        pl.kernel,
        out_type=jax.ShapeDtypeStruct((2 * N, f2), jnp.float32),
        mesh=mesh,
        compiler_params=pltpu.CompilerParams(use_tc_tiling_on_sc=False),
        scratch_types=[
            pltpu.VMEM_SHARED((N, f2), jnp.float32),
            pltpu.VMEM((EB,), jnp.int32),
            pltpu.VMEM((EB,), jnp.int32),
            pltpu.VMEM((EB, f2), jnp.float32),
            pltpu.SemaphoreType.DMA,
        ],
    )
    def k(z_hbm, src2_hbm, dst_hbm, out_hbm, acc, sidx, didx, rows, gsem):
        c = lax.axis_index("c")
        t = lax.axis_index("s")

        def init_body(i, _):
            r0 = (t + i * NTILE) * RB
            pltpu.sync_copy(z_hbm.at[pl.ds(c * N + r0, RB)],
                            rows.at[pl.ds(0, RB)])
            pltpu.sync_copy(rows.at[pl.ds(0, RB)], acc.at[pl.ds(r0, RB)])
            return 0

        n_init = (N_ROW_CHUNKS - t + NTILE - 1) // NTILE
        lax.fori_loop(0, n_init, init_body, 0)
        plsc.subcore_barrier()

        def edge_body(i, _):
            base = (t + i * NTILE) * EB
            pltpu.sync_copy(src2_hbm.at[pl.ds(c * E + base, EB)], sidx)
            pltpu.sync_copy(dst_hbm.at[pl.ds(base, EB)], didx)
            pltpu.async_copy(z_hbm.at[sidx], rows, gsem).wait()
            pltpu.sync_copy(rows, acc.at[didx], add=True)
            return 0

        n_edge = (N_EDGE_CHUNKS - t + NTILE - 1) // NTILE
        lax.fori_loop(0, n_edge, edge_body, 0)
        plsc.subcore_barrier()

        def drain_body(i, _):
            r0 = (t + i * NTILE) * RB
            pltpu.sync_copy(acc.at[pl.ds(r0, RB)], rows.at[pl.ds(0, RB)])
            pltpu.sync_copy(rows.at[pl.ds(0, RB)],
                            out_hbm.at[pl.ds(c * N + r0, RB)])
            return 0

        lax.fori_loop(0, n_init, drain_body, 0)

    return k(zs, src2, dst)


TC_RB = 400


def _tc_layer1(s1, W1, b1, W2):
    f2 = W2.shape[1] // 2

    def body(s_ref, w1_ref, b1_ref, w2_ref, o_ref):
        h = jnp.concatenate([s_ref[0], s_ref[1]], axis=1)
        h1 = jax.nn.relu(jnp.dot(h, w1_ref[...],
                                 preferred_element_type=jnp.float32) + b1_ref[...])
        z = jnp.dot(h1, w2_ref[...], preferred_element_type=jnp.float32)
        o_ref[0] = z[:, :f2]
        o_ref[1] = z[:, f2:]

    return pl.pallas_call(
        body,
        grid=(N // TC_RB,),
        in_specs=[
            pl.BlockSpec((2, TC_RB, s1.shape[2]), lambda r: (0, r, 0)),
            pl.BlockSpec(W1.shape, lambda r: (0, 0)),
            pl.BlockSpec(b1.shape, lambda r: (0, 0)),
            pl.BlockSpec(W2.shape, lambda r: (0, 0)),
        ],
        out_specs=pl.BlockSpec((2, TC_RB, f2), lambda r: (0, r, 0)),
        out_shape=jax.ShapeDtypeStruct((2, N, f2), jnp.float32),
    )(s1, W1, b1, W2)


def _tc_mid(s, b, W):
    f2 = W.shape[1] // 2

    def body(s_ref, b_ref, w_ref, o_ref):
        h = jax.nn.relu(jnp.concatenate([s_ref[0], s_ref[1]], axis=1)
                        + b_ref[...])
        z = jnp.dot(h, w_ref[...], preferred_element_type=jnp.float32)
        o_ref[0] = z[:, :f2]
        o_ref[1] = z[:, f2:]

    return pl.pallas_call(
        body,
        grid=(N // TC_RB,),
        in_specs=[
            pl.BlockSpec((2, TC_RB, s.shape[2]), lambda r: (0, r, 0)),
            pl.BlockSpec(b.shape, lambda r: (0, 0)),
            pl.BlockSpec(W.shape, lambda r: (0, 0)),
        ],
        out_specs=pl.BlockSpec((2, TC_RB, f2), lambda r: (0, r, 0)),
        out_shape=jax.ShapeDtypeStruct((2, N, f2), jnp.float32),
    )(s, b, W)


def _tc_logsoftmax(s5, b5, n_cls):
    def body(s_ref, b_ref, o_ref):
        y = jnp.concatenate([s_ref[0], s_ref[1]], axis=1)[:, :n_cls] + b_ref[...]
        m = jnp.max(y, axis=1, keepdims=True)
        e = jnp.exp(y - m)
        o_ref[...] = y - m - jnp.log(jnp.sum(e, axis=1, keepdims=True))

    return pl.pallas_call(
        body,
        grid=(N // TC_RB,),
        in_specs=[
            pl.BlockSpec((2, TC_RB, s5.shape[2]), lambda r: (0, r, 0)),
            pl.BlockSpec(b5.shape, lambda r: (0, 0)),
        ],
        out_specs=pl.BlockSpec((TC_RB, n_cls), lambda r: (r, 0)),
        out_shape=jax.ShapeDtypeStruct((N, n_cls), jnp.float32),
    )(s5, b5)


def kernel(x, edge_index, W1, b1, W2, b2, W3, b3, W4, b4, W5, b5):
    n_cls = W5.shape[1]
    src = edge_index[0]
    dst = edge_index[1]
    src2 = jnp.concatenate([src, src + N])

    W5p = jnp.pad(W5, ((0, 0), (0, 64 - n_cls)))

    b1r = b1.reshape(1, -1)
    b2r = b2.reshape(1, -1)
    b3r = b3.reshape(1, -1)
    b4r = b4.reshape(1, -1)
    b5r = b5.reshape(1, -1)

    x2 = jnp.concatenate([x[:, :64], x[:, 64:]], axis=0)
    s1 = _sc_aggregate(x2, src2, dst, 64)
    z2 = _tc_layer1(s1.reshape(2, N, 64), W1, b1r, W2)

    s2 = _sc_aggregate(z2.reshape(2 * N, 128), src2, dst, 128)
    z3 = _tc_mid(s2.reshape(2, N, 128), b2r, W3)

    s3 = _sc_aggregate(z3.reshape(2 * N, 128), src2, dst, 128)
    z4 = _tc_mid(s3.reshape(2, N, 128), b3r, W4)

    s4 = _sc_aggregate(z4.reshape(2 * N, 128), src2, dst, 128)
    z5 = _tc_mid(s4.reshape(2, N, 128), b4r, W5p)

    s5 = _sc_aggregate(z5.reshape(2 * N, 32), src2, dst, 32)
    return _tc_logsoftmax(s5.reshape(2, N, 32), b5r, n_cls)


# speedup vs baseline: 4.8761x; 1.0803x over previous
import functools

import jax
import jax.numpy as jnp
from jax import lax
from jax.experimental import pallas as pl
from jax.experimental.pallas import tpu as pltpu
from jax.experimental.pallas import tpu_sc as plsc

N = 10000
E = 320000
NSC = 2
NTILE = 16
EB = 128
N_EDGE_CHUNKS = E // EB
DUMMY = 64
ROWS_PER_TILE = N // NTILE


CH = 8
P_CHUNKS = 2560
N_STEPS = P_CHUNKS // (NTILE * CH)


def _sc_aggregate(zs, src2p, dstp, f2):
    mesh = plsc.VectorSubcoreMesh(core_axis_name="c", subcore_axis_name="s")

    @functools.partial(
        pl.kernel,
        out_type=jax.ShapeDtypeStruct((2 * N, f2), jnp.float32),
        mesh=mesh,
        compiler_params=pltpu.CompilerParams(use_tc_tiling_on_sc=False),
        scratch_types=[
            pltpu.VMEM_SHARED((N + DUMMY, f2), jnp.float32),
            pltpu.VMEM((2, CH, EB), jnp.int32),
            pltpu.VMEM((2, CH, EB), jnp.int32),
            pltpu.VMEM((2, EB, f2), jnp.float32),
            pltpu.SemaphoreType.DMA,
            pltpu.SemaphoreType.DMA,
        ],
    )
    def kfn(z_hbm, src2_hbm, dst_hbm, out_hbm, acc, sidx, didx, rows, g0, g1):
        c = lax.axis_index("c")
        t = lax.axis_index("s")
        gsems = (g0, g1)

        def load_idx(step, set_i):
            base = (step * NTILE + t) * CH
            pltpu.sync_copy(src2_hbm.at[c, pl.ds(base, CH)], sidx.at[set_i])
            pltpu.sync_copy(dst_hbm.at[pl.ds(base, CH)], didx.at[set_i])

        def fire(set_i, j, rb):
            pltpu.async_copy(z_hbm.at[sidx.at[set_i, j]], rows.at[rb],
                             gsems[rb])

        def wait(set_i, j, rb):
            pltpu.make_async_copy(z_hbm.at[sidx.at[set_i, j]], rows.at[rb],
                                  gsems[rb]).wait()

        def scat(set_i, j, rb):
            pltpu.sync_copy(rows.at[rb], acc.at[didx.at[set_i, j]], add=True)

        load_idx(0, 0)
        fire(0, 0, 0)

        r0 = t * ROWS_PER_TILE
        pltpu.sync_copy(z_hbm.at[pl.ds(c * N + r0, ROWS_PER_TILE)],
                        acc.at[pl.ds(r0, ROWS_PER_TILE)])
        plsc.subcore_barrier()

        def body(i, _):
            for set_i in (0, 1):
                s = 2 * i + set_i
                for j in range(CH):
                    rb = j % 2
                    if j == 0:
                        @pl.when(s + 1 < N_STEPS)
                        def _():
                            load_idx(s + 1, 1 - set_i)
                    if j < CH - 1:
                        fire(set_i, j + 1, 1 - rb)
                    else:
                        @pl.when(s + 1 < N_STEPS)
                        def _():
                            fire(1 - set_i, 0, 1 - rb)
                    wait(set_i, j, rb)
                    scat(set_i, j, rb)
            return 0

        lax.fori_loop(0, N_STEPS // 2, body, 0)
        plsc.subcore_barrier()

        pltpu.sync_copy(acc.at[pl.ds(r0, ROWS_PER_TILE)],
                        out_hbm.at[pl.ds(c * N + r0, ROWS_PER_TILE)])

    return kfn(zs, src2p, dstp)


def _pad_edges(src, dst, p_chunks):
    pe = p_chunks * EB
    pad = pe - E
    srcp = jnp.concatenate([src, jnp.zeros((pad,), jnp.int32)])
    src2p = jnp.stack([srcp, srcp + N]).reshape(2, p_chunks, EB)
    dstp = jnp.concatenate(
        [dst, N + (jnp.arange(pad, dtype=jnp.int32) % DUMMY)]
    ).reshape(p_chunks, EB)
    return src2p, dstp


TC_RB = 400


def _tc_layer1(s1, W1, b1, W2):
    f2 = W2.shape[1] // 2

    def body(s_ref, w1_ref, b1_ref, w2_ref, o_ref):
        h = jnp.concatenate([s_ref[0], s_ref[1]], axis=1)
        h1 = jax.nn.relu(jnp.dot(h, w1_ref[...],
                                 preferred_element_type=jnp.float32) + b1_ref[...])
        z = jnp.dot(h1, w2_ref[...], preferred_element_type=jnp.float32)
        o_ref[0] = z[:, :f2]
        o_ref[1] = z[:, f2:]

    return pl.pallas_call(
        body,
        grid=(N // TC_RB,),
        in_specs=[
            pl.BlockSpec((2, TC_RB, s1.shape[2]), lambda r: (0, r, 0)),
            pl.BlockSpec(W1.shape, lambda r: (0, 0)),
            pl.BlockSpec(b1.shape, lambda r: (0, 0)),
            pl.BlockSpec(W2.shape, lambda r: (0, 0)),
        ],
        out_specs=pl.BlockSpec((2, TC_RB, f2), lambda r: (0, r, 0)),
        out_shape=jax.ShapeDtypeStruct((2, N, f2), jnp.float32),
    )(s1, W1, b1, W2)


def _tc_mid(s, b, W):
    f2 = W.shape[1] // 2

    def body(s_ref, b_ref, w_ref, o_ref):
        h = jax.nn.relu(jnp.concatenate([s_ref[0], s_ref[1]], axis=1)
                        + b_ref[...])
        z = jnp.dot(h, w_ref[...], preferred_element_type=jnp.float32)
        o_ref[0] = z[:, :f2]
        o_ref[1] = z[:, f2:]

    return pl.pallas_call(
        body,
        grid=(N // TC_RB,),
        in_specs=[
            pl.BlockSpec((2, TC_RB, s.shape[2]), lambda r: (0, r, 0)),
            pl.BlockSpec(b.shape, lambda r: (0, 0)),
            pl.BlockSpec(W.shape, lambda r: (0, 0)),
        ],
        out_specs=pl.BlockSpec((2, TC_RB, f2), lambda r: (0, r, 0)),
        out_shape=jax.ShapeDtypeStruct((2, N, f2), jnp.float32),
    )(s, b, W)


def _tc_logsoftmax(s5, b5, n_cls):
    def body(s_ref, b_ref, o_ref):
        y = jnp.concatenate([s_ref[0], s_ref[1]], axis=1)[:, :n_cls] + b_ref[...]
        m = jnp.max(y, axis=1, keepdims=True)
        e = jnp.exp(y - m)
        o_ref[...] = y - m - jnp.log(jnp.sum(e, axis=1, keepdims=True))

    return pl.pallas_call(
        body,
        grid=(N // TC_RB,),
        in_specs=[
            pl.BlockSpec((2, TC_RB, s5.shape[2]), lambda r: (0, r, 0)),
            pl.BlockSpec(b5.shape, lambda r: (0, 0)),
        ],
        out_specs=pl.BlockSpec((TC_RB, n_cls), lambda r: (r, 0)),
        out_shape=jax.ShapeDtypeStruct((N, n_cls), jnp.float32),
    )(s5, b5)


def kernel(x, edge_index, W1, b1, W2, b2, W3, b3, W4, b4, W5, b5):
    n_cls = W5.shape[1]
    src = edge_index[0]
    dst = edge_index[1]

    src2p, dstp = _pad_edges(src, dst, P_CHUNKS)

    W5p = jnp.pad(W5, ((0, 0), (0, 64 - n_cls)))

    b1r = b1.reshape(1, -1)
    b2r = b2.reshape(1, -1)
    b3r = b3.reshape(1, -1)
    b4r = b4.reshape(1, -1)
    b5r = b5.reshape(1, -1)

    x2 = jnp.concatenate([x[:, :64], x[:, 64:]], axis=0)
    s1 = _sc_aggregate(x2, src2p, dstp, 64)
    z2 = _tc_layer1(s1.reshape(2, N, 64), W1, b1r, W2)

    s2 = _sc_aggregate(z2.reshape(2 * N, 128), src2p, dstp, 128)
    z3 = _tc_mid(s2.reshape(2, N, 128), b2r, W3)

    s3 = _sc_aggregate(z3.reshape(2 * N, 128), src2p, dstp, 128)
    z4 = _tc_mid(s3.reshape(2, N, 128), b3r, W4)

    s4 = _sc_aggregate(z4.reshape(2 * N, 128), src2p, dstp, 128)
    z5 = _tc_mid(s4.reshape(2, N, 128), b4r, W5p)

    s5 = _sc_aggregate(z5.reshape(2 * N, 32), src2p, dstp, 32)
    return _tc_logsoftmax(s5.reshape(2, N, 32), b5r, n_cls)
